# baseline (speedup 1.0000x reference)
import jax
import jax.numpy as jnp
from jax import lax
from jax.experimental import pallas as pl
from jax.experimental.pallas import tpu as pltpu
from jax.experimental.pallas import tpu_sc as plsc

B, L, C = 16, 2048, 1024
T = 2 * L
NC, NS = 2, 16
HALF = L // 2
CH = 16
GN = 24
NCH = HALF // CH
LANES = 16
CV = C // LANES
INV = 1.0 / float(T - 1)


def _weights(m, p_pad):
    mf = m.astype(jnp.float32)
    zero = jnp.float32(0.0)
    alpha = mf * INV
    w_prev = jnp.where(m >= p_pad + 1, alpha, zero)
    w_cur_e = jnp.where(m >= p_pad, 1.0 - alpha, zero)
    w_cur_o = jnp.where(m >= p_pad, (mf + float(L)) * INV, zero)
    w_next = jnp.where(m >= p_pad - 1, (float(L - 1) - mf) * INV, zero)
    return (jnp.broadcast_to(w_prev, (LANES,)),
            jnp.broadcast_to(w_cur_e, (LANES,)),
            jnp.broadcast_to(w_cur_o, (LANES,)),
            jnp.broadcast_to(w_next, (LANES,)))


def _sc_body(ctx_hbm, mask_hbm, out_hbm, tm_hbm,
             mask_v, ext_v, tm_v, buf0, buf1, ob0, ob1,
             sg0, sg1, sw0, sw1):
    wid = lax.axis_index("s") * NC + lax.axis_index("c")
    b = wid // 2
    half = wid % 2
    base = b * L

    pltpu.sync_copy(mask_hbm.at[b], mask_v)

    basev = jnp.broadcast_to(base, (LANES,))

    def _init_ext(j, carry):
        ext_v[pl.ds(j * LANES, LANES)] = basev
        return carry

    lax.fori_loop(0, (L + 2 * GN) // LANES, _init_ext, jnp.int32(0))

    def _count(j, tot):
        return tot + jnp.sum(mask_v[pl.ds(j * LANES, LANES)])

    nvalid = lax.fori_loop(0, L // LANES, _count, jnp.int32(0))
    p_pad = jnp.int32(L) - nvalid

    lane = lax.iota(jnp.int32, LANES)

    def _scatter(j, cum):
        mv = mask_v[pl.ds(j * LANES, LANES)]
        csum = jnp.cumsum(mv) + cum
        jloc = lane + j * LANES
        dest = jnp.where(mv > 0, p_pad - 1 + csum, jloc - csum)
        plsc.store_scatter(ext_v, [dest + 1], jloc + base)
        return cum + jnp.sum(mv)

    lax.fori_loop(0, L // LANES, _scatter, jnp.int32(0))

    two_p = 2 * p_pad

    def _tmrow(j, carry):
        g = lane + (j * LANES + half * (2 * HALF))
        tm_v[pl.ds(j * LANES, LANES)] = (g >= two_p).astype(jnp.int32)
        return carry

    lax.fori_loop(0, (2 * HALF) // LANES, _tmrow, jnp.int32(0))
    pltpu.sync_copy(tm_v, tm_hbm.at[pl.ds(b * T + half * (2 * HALF), 2 * HALF)])

    zv = jnp.zeros((LANES,), jnp.float32)

    def _zbuf(j, carry):
        r = j // CV
        o = pl.ds((j % CV) * LANES, LANES)
        buf0[r, o] = zv
        buf1[r, o] = zv
        return carry

    lax.fori_loop(0, GN * CV, _zbuf, jnp.int32(0))

    bufs = (buf0, buf1)
    obufs = (ob0, ob1)
    gsems = (sg0, sg1)
    wsems = (sw0, sw1)
    s_max = (2 * (NCH - 1) + 1) * CH

    def _s_of(k):
        return jnp.minimum((2 * k + half) * CH, s_max)

    def _active(s_p):
        return s_p + CH > p_pad - 1

    def _gather_desc(k, slot):
        s_p = _s_of(k)
        idx = ext_v.at[pl.ds(s_p, GN)]
        return pltpu.make_async_copy(ctx_hbm.at[idx], bufs[slot],
                                     gsems[slot])

    def _start_gather(k, slot):
        @pl.when(_active(_s_of(k)))
        def _():
            _gather_desc(k, slot).start()

    def _wait_gather(k, slot):
        @pl.when(_active(_s_of(k)))
        def _():
            _gather_desc(k, slot).wait()

    for sl in range(2):
        _start_gather(jnp.int32(sl), sl)

    def _chunk(k2, carry, sl):
        k = 2 * k2 + sl
        s_p = (2 * k + half) * CH
        buf = bufs[sl]
        obuf = obufs[sl]
        _wait_gather(k, sl)

        @pl.when(k2 >= 1)
        def _():
            pltpu.make_async_copy(obuf, out_hbm.at[pl.ds(0, 2 * CH)],
                                  wsems[sl]).wait()

        def _pair(ip, c2):
            i = 2 * ip
            m0 = s_p + i
            wp0, we0, wo0, wn0 = _weights(m0, p_pad)
            wp1, we1, wo1, wn1 = _weights(m0 + 1, p_pad)

            @plsc.parallel_loop(0, CV, unroll=8)
            def _col(c):
                off = pl.ds(c * LANES, LANES)
                a = buf[i, off]
                bq = buf[i + 1, off]
                cq = buf[i + 2, off]
                dq = buf[i + 3, off]
                obuf[2 * i, off] = wp0 * a + we0 * bq
                obuf[2 * i + 1, off] = wo0 * bq + wn0 * cq
                obuf[2 * i + 2, off] = wp1 * bq + we1 * cq
                obuf[2 * i + 3, off] = wo1 * cq + wn1 * dq

            return c2

        lax.fori_loop(0, CH // 2, _pair, jnp.int32(0))

        pltpu.async_copy(obuf, out_hbm.at[pl.ds(b * T + 2 * s_p, 2 * CH)],
                         wsems[sl])
        _start_gather(k + 2, sl)
        return carry

    def _ring(k2, carry):
        carry = _chunk(k2, carry, 0)
        carry = _chunk(k2, carry, 1)
        return carry

    lax.fori_loop(0, NCH // 2, _ring, jnp.int32(0))
    for sl in range(2):
        _wait_gather(jnp.int32(NCH + sl), sl)
        pltpu.make_async_copy(obufs[sl], out_hbm.at[pl.ds(0, 2 * CH)],
                              wsems[sl]).wait()


def _sc_interp(ctx_flat, mask):
    mesh = plsc.VectorSubcoreMesh(core_axis_name="c", subcore_axis_name="s")
    return pl.kernel(
        _sc_body,
        out_type=[
            jax.ShapeDtypeStruct((B * T, C), jnp.float32),
            jax.ShapeDtypeStruct((B * T,), jnp.int32),
        ],
        mesh=mesh,
        compiler_params=pltpu.CompilerParams(needs_layout_passes=False),
        scratch_types=[
            pltpu.VMEM((L,), jnp.int32),
            pltpu.VMEM((L + 2 * GN,), jnp.int32),
            pltpu.VMEM((2 * HALF,), jnp.int32),
            pltpu.VMEM((GN, C), jnp.float32),
            pltpu.VMEM((GN, C), jnp.float32),
            pltpu.VMEM((2 * CH, C), jnp.float32),
            pltpu.VMEM((2 * CH, C), jnp.float32),
            pltpu.SemaphoreType.DMA,
            pltpu.SemaphoreType.DMA,
            pltpu.SemaphoreType.DMA,
            pltpu.SemaphoreType.DMA,
        ],
    )(ctx_flat, mask)


def kernel(context, target_length, context_mask):
    del target_length
    ctx_flat = context.reshape(B * L, C)
    out_flat, tm_flat = _sc_interp(ctx_flat, context_mask)
    out = out_flat.reshape(B, T, C)
    tmask = tm_flat.reshape(B, T).astype(bool)
    return out, tmask


# speedup vs baseline: 1.0356x; 1.0356x over previous
import jax
import jax.numpy as jnp
from jax import lax
from jax.experimental import pallas as pl
from jax.experimental.pallas import tpu as pltpu
from jax.experimental.pallas import tpu_sc as plsc

B, L, C = 16, 2048, 1024
T = 2 * L
NC, NS = 2, 16
HALF = L // 2
CH = 16
GN = 24
NCH = HALF // CH
LANES = 16
CV = C // LANES
INV = 1.0 / float(T - 1)


def _weights(m, p_pad):
    mf = m.astype(jnp.float32)
    zero = jnp.float32(0.0)
    alpha = mf * INV
    w_prev = jnp.where(m >= p_pad + 1, alpha, zero)
    w_cur_e = jnp.where(m >= p_pad, 1.0 - alpha, zero)
    w_cur_o = jnp.where(m >= p_pad, (mf + float(L)) * INV, zero)
    w_next = jnp.where(m >= p_pad - 1, (float(L - 1) - mf) * INV, zero)
    return (jnp.broadcast_to(w_prev, (LANES,)),
            jnp.broadcast_to(w_cur_e, (LANES,)),
            jnp.broadcast_to(w_cur_o, (LANES,)),
            jnp.broadcast_to(w_next, (LANES,)))


def _sc_body(ctx_hbm, mask_hbm, out_hbm, tm_hbm,
             mask_v, ext_v, tm_v, buf0, buf1, ob0, ob1,
             sg0, sg1, sw0, sw1):
    wid = lax.axis_index("s") * NC + lax.axis_index("c")
    b = wid // 2
    half = wid % 2
    base = b * L

    pltpu.sync_copy(mask_hbm.at[b], mask_v)

    basev = jnp.broadcast_to(base, (LANES,))

    @plsc.parallel_loop(0, (L + 2 * GN) // LANES, unroll=8)
    def _init_ext(j):
        ext_v[pl.ds(j * LANES, LANES)] = basev

    def _count(j, tot):
        return tot + jnp.sum(mask_v[pl.ds(j * LANES, LANES)])

    nvalid = lax.fori_loop(0, L // LANES, _count, jnp.int32(0))
    p_pad = jnp.int32(L) - nvalid

    lane = lax.iota(jnp.int32, LANES)

    def _scatter(j, cum):
        mv = mask_v[pl.ds(j * LANES, LANES)]
        csum = jnp.cumsum(mv) + cum
        jloc = lane + j * LANES
        dest = jnp.where(mv > 0, p_pad - 1 + csum, jloc - csum)
        plsc.store_scatter(ext_v, [dest + 1], jloc + base)
        return cum + jnp.sum(mv)

    lax.fori_loop(0, L // LANES, _scatter, jnp.int32(0))

    two_p = 2 * p_pad

    @plsc.parallel_loop(0, (2 * HALF) // LANES, unroll=8)
    def _tmrow(j):
        g = lane + (j * LANES + half * (2 * HALF))
        tm_v[pl.ds(j * LANES, LANES)] = (g >= two_p).astype(jnp.int32)
    pltpu.sync_copy(tm_v, tm_hbm.at[pl.ds(b * T + half * (2 * HALF), 2 * HALF)])

    zv = jnp.zeros((LANES,), jnp.float32)

    @plsc.parallel_loop(0, GN * CV, unroll=8)
    def _zbuf(j):
        r = j // CV
        o = pl.ds((j % CV) * LANES, LANES)
        buf0[r, o] = zv
        buf1[r, o] = zv

    bufs = (buf0, buf1)
    obufs = (ob0, ob1)
    gsems = (sg0, sg1)
    wsems = (sw0, sw1)
    s_max = (2 * (NCH - 1) + 1) * CH

    def _s_of(k):
        return jnp.minimum((2 * k + half) * CH, s_max)

    def _active(s_p):
        return s_p + CH > p_pad - 1

    def _gather_desc(k, slot):
        s_p = _s_of(k)
        idx = ext_v.at[pl.ds(s_p, GN)]
        return pltpu.make_async_copy(ctx_hbm.at[idx], bufs[slot],
                                     gsems[slot])

    def _start_gather(k, slot):
        @pl.when(_active(_s_of(k)))
        def _():
            _gather_desc(k, slot).start()

    def _wait_gather(k, slot):
        @pl.when(_active(_s_of(k)))
        def _():
            _gather_desc(k, slot).wait()

    for sl in range(2):
        _start_gather(jnp.int32(sl), sl)

    def _chunk(k2, carry, sl):
        k = 2 * k2 + sl
        s_p = (2 * k + half) * CH
        buf = bufs[sl]
        obuf = obufs[sl]
        _wait_gather(k, sl)

        @pl.when(k2 >= 1)
        def _():
            pltpu.make_async_copy(obuf, out_hbm.at[pl.ds(0, 2 * CH)],
                                  wsems[sl]).wait()

        def _pair(ip, c2):
            i = 2 * ip
            m0 = s_p + i
            wp0, we0, wo0, wn0 = _weights(m0, p_pad)
            wp1, we1, wo1, wn1 = _weights(m0 + 1, p_pad)

            @plsc.parallel_loop(0, CV, unroll=8)
            def _col(c):
                off = pl.ds(c * LANES, LANES)
                a = buf[i, off]
                bq = buf[i + 1, off]
                cq = buf[i + 2, off]
                dq = buf[i + 3, off]
                obuf[2 * i, off] = wp0 * a + we0 * bq
                obuf[2 * i + 1, off] = wo0 * bq + wn0 * cq
                obuf[2 * i + 2, off] = wp1 * bq + we1 * cq
                obuf[2 * i + 3, off] = wo1 * cq + wn1 * dq

            return c2

        lax.fori_loop(0, CH // 2, _pair, jnp.int32(0))

        pltpu.async_copy(obuf, out_hbm.at[pl.ds(b * T + 2 * s_p, 2 * CH)],
                         wsems[sl])
        _start_gather(k + 2, sl)
        return carry

    def _ring(k2, carry):
        carry = _chunk(k2, carry, 0)
        carry = _chunk(k2, carry, 1)
        return carry

    lax.fori_loop(0, NCH // 2, _ring, jnp.int32(0))
    for sl in range(2):
        _wait_gather(jnp.int32(NCH + sl), sl)
        pltpu.make_async_copy(obufs[sl], out_hbm.at[pl.ds(0, 2 * CH)],
                              wsems[sl]).wait()


def _sc_interp(ctx_flat, mask):
    mesh = plsc.VectorSubcoreMesh(core_axis_name="c", subcore_axis_name="s")
    return pl.kernel(
        _sc_body,
        out_type=[
            jax.ShapeDtypeStruct((B * T, C), jnp.float32),
            jax.ShapeDtypeStruct((B * T,), jnp.int32),
        ],
        mesh=mesh,
        compiler_params=pltpu.CompilerParams(needs_layout_passes=False),
        scratch_types=[
            pltpu.VMEM((L,), jnp.int32),
            pltpu.VMEM((L + 2 * GN,), jnp.int32),
            pltpu.VMEM((2 * HALF,), jnp.int32),
            pltpu.VMEM((GN, C), jnp.float32),
            pltpu.VMEM((GN, C), jnp.float32),
            pltpu.VMEM((2 * CH, C), jnp.float32),
            pltpu.VMEM((2 * CH, C), jnp.float32),
            pltpu.SemaphoreType.DMA,
            pltpu.SemaphoreType.DMA,
            pltpu.SemaphoreType.DMA,
            pltpu.SemaphoreType.DMA,
        ],
    )(ctx_flat, mask)


def kernel(context, target_length, context_mask):
    del target_length
    ctx_flat = context.reshape(B * L, C)
    out_flat, tm_flat = _sc_interp(ctx_flat, context_mask)
    out = out_flat.reshape(B, T, C)
    tmask = tm_flat.reshape(B, T).astype(bool)
    return out, tmask
